# baseline (speedup 1.0000x reference)
import functools

import jax
import jax.numpy as jnp
from jax import lax
from jax.experimental import pallas as pl
from jax.experimental.pallas import tpu as pltpu
from jax.experimental.pallas import tpu_sc as plsc

N_NODES = 10000
N_EDGES = 640000
IN_CH = 116
HID = 64

NC = 2
NS = 16
NW = NC * NS
EDGES_PER_W = N_EDGES // NW
DEG_CHUNK = 2000
AGG_CHUNK = 200
ROW_CHUNK = 640

_mesh = plsc.VectorSubcoreMesh(core_axis_name="c", subcore_axis_name="s")
_sc_params = pltpu.CompilerParams(use_tc_tiling_on_sc=False)


def _mm_body(x_ref, w_ref, h_ref):
    h_ref[...] = jnp.dot(x_ref[...], w_ref[...],
                         preferred_element_type=jnp.float32)


def _matmul(x, W1):
    return pl.pallas_call(
        _mm_body,
        grid=(10,),
        in_specs=[
            pl.BlockSpec((N_NODES // 10, IN_CH), lambda i: (i, 0)),
            pl.BlockSpec((IN_CH, HID), lambda i: (0, 0)),
        ],
        out_specs=pl.BlockSpec((N_NODES // 10, HID), lambda i: (i, 0)),
        out_shape=jax.ShapeDtypeStruct((N_NODES, HID), jnp.float32),
    )(x, W1)


@functools.partial(
    pl.kernel,
    out_type=jax.ShapeDtypeStruct((NC, N_NODES), jnp.float32),
    mesh=_mesh,
    scratch_types=[
        pltpu.VMEM((DEG_CHUNK,), jnp.int32),
        pltpu.VMEM((DEG_CHUNK,), jnp.float32),
        pltpu.VMEM((DEG_CHUNK,), jnp.float32),
        pltpu.VMEM_SHARED((N_NODES,), jnp.float32),
    ],
    compiler_params=_sc_params,
)
def _deg_kernel(adj_hbm, deg_out, idx_v, ones_v, zeros_v, deg_shared):
    cid = lax.axis_index("c")
    sid = lax.axis_index("s")
    wid = cid * NS + sid

    @pl.loop(0, DEG_CHUNK, step=16)
    def _(i):
        ones_v[pl.ds(i, 16)] = jnp.full((16,), 1.0, jnp.float32)
        zeros_v[pl.ds(i, 16)] = jnp.zeros((16,), jnp.float32)

    @pl.when(sid == 0)
    def _():
        for j in range(N_NODES // DEG_CHUNK):
            pltpu.sync_copy(zeros_v, deg_shared.at[pl.ds(j * DEG_CHUNK,
                                                         DEG_CHUNK)])

    plsc.subcore_barrier()

    base = wid * EDGES_PER_W
    for i in range(EDGES_PER_W // DEG_CHUNK):
        pltpu.sync_copy(adj_hbm.at[1, pl.ds(base + i * DEG_CHUNK, DEG_CHUNK)],
                        idx_v)
        pltpu.sync_copy(ones_v, deg_shared.at[idx_v], add=True)

    plsc.subcore_barrier()

    @pl.when(sid == 0)
    def _():
        pltpu.sync_copy(deg_shared, deg_out.at[cid])


N_CHUNKS = EDGES_PER_W // AGG_CHUNK
SCALE_BLK = 80


def _rsqrt_newton(d):
    i = lax.bitcast_convert_type(d, jnp.int32)
    y = lax.bitcast_convert_type(
        jnp.int32(0x5F3759DF) - lax.shift_right_logical(i, 1), jnp.float32)
    for _ in range(3):
        y = y * (1.5 - 0.5 * d * y * y)
    return y


@functools.partial(
    pl.kernel,
    out_type=jax.ShapeDtypeStruct((NC, N_NODES, HID), jnp.float32),
    mesh=_mesh,
    scratch_types=[
        pltpu.VMEM((AGG_CHUNK,), jnp.int32),
        pltpu.VMEM((AGG_CHUNK,), jnp.int32),
        pltpu.VMEM((AGG_CHUNK,), jnp.int32),
        pltpu.VMEM((AGG_CHUNK,), jnp.int32),
        pltpu.VMEM((AGG_CHUNK, HID), jnp.float32),
        pltpu.VMEM((AGG_CHUNK, HID), jnp.float32),
        pltpu.VMEM((SCALE_BLK, HID), jnp.float32),
        pltpu.VMEM((ROW_CHUNK,), jnp.float32),
        pltpu.VMEM((ROW_CHUNK,), jnp.float32),
        pltpu.VMEM_SHARED((N_NODES, HID), jnp.float32),
        pltpu.VMEM_SHARED((N_NODES, HID), jnp.float32),
        pltpu.SemaphoreType.DMA,
        pltpu.SemaphoreType.DMA,
        pltpu.SemaphoreType.DMA,
        pltpu.SemaphoreType.DMA,
        pltpu.SemaphoreType.DMA,
    ],
    compiler_params=_sc_params,
)
def _agg_kernel(h_hbm, adj_hbm, deg_hbm, acc_out,
                si0_v, si1_v, di0_v, di1_v, rows0_v, rows1_v, blk_v,
                dv0_v, dv1_v, acc_shared, hs_shared,
                sem_g0, sem_g1, sem_i0, sem_i1, sem_h):
    cid = lax.axis_index("c")
    sid = lax.axis_index("s")
    wid = cid * NS + sid
    r0 = sid * ROW_CHUNK

    def _setup(nloc):
        nblk = nloc // SCALE_BLK
        hcp = pltpu.async_copy(h_hbm.at[pl.ds(r0, nloc)],
                               hs_shared.at[pl.ds(r0, nloc)], sem_h)

        @pl.loop(0, SCALE_BLK)
        def _(r):
            @pl.loop(0, HID, step=16)
            def _(c):
                blk_v[r, pl.ds(c, 16)] = jnp.zeros((16,), jnp.float32)

        for j in range(nblk):
            pltpu.sync_copy(blk_v,
                            acc_shared.at[pl.ds(r0 + j * SCALE_BLK,
                                                SCALE_BLK)])

        pltpu.sync_copy(deg_hbm.at[0, pl.ds(r0, nloc)], dv0_v.at[pl.ds(0, nloc)])
        pltpu.sync_copy(deg_hbm.at[1, pl.ds(r0, nloc)], dv1_v.at[pl.ds(0, nloc)])

        @pl.loop(0, nloc, step=16)
        def _(i):
            d = dv0_v[pl.ds(i, 16)] + dv1_v[pl.ds(i, 16)] + 1.0
            dv0_v[pl.ds(i, 16)] = _rsqrt_newton(d)

        hcp.wait()
        for j in range(nblk):
            pltpu.sync_copy(hs_shared.at[pl.ds(r0 + j * SCALE_BLK, SCALE_BLK)],
                            blk_v)

            @pl.loop(0, SCALE_BLK, step=16)
            def _(rb):
                dvec = dv0_v[pl.ds(j * SCALE_BLK + rb, 16)]
                for k in range(16):
                    s = dvec[k]
                    for c in range(0, HID, 16):
                        blk_v[rb + k, pl.ds(c, 16)] = (
                            blk_v[rb + k, pl.ds(c, 16)] * s)

            pltpu.sync_copy(blk_v,
                            hs_shared.at[pl.ds(r0 + j * SCALE_BLK, SCALE_BLK)])

    @pl.when(sid < NS - 1)
    def _():
        _setup(ROW_CHUNK)

    @pl.when(sid == NS - 1)
    def _():
        _setup(N_NODES - (NS - 1) * ROW_CHUNK)

    plsc.subcore_barrier()

    si = (si0_v, si1_v)
    di = (di0_v, di1_v)
    rows = (rows0_v, rows1_v)
    gsems = (sem_g0, sem_g1)
    isems = (sem_i0, sem_i1)
    ebase = wid * EDGES_PER_W

    def _stage(j):
        b = j % 2
        off = ebase + j * AGG_CHUNK
        return (
            pltpu.async_copy(adj_hbm.at[0, pl.ds(off, AGG_CHUNK)], si[b],
                             isems[b]),
            pltpu.async_copy(adj_hbm.at[1, pl.ds(off, AGG_CHUNK)], di[b],
                             isems[b]),
        )

    def _gather(j):
        return pltpu.async_copy(hs_shared.at[si[j % 2]], rows[j % 2],
                                gsems[j % 2])

    st = _stage(0)
    st[0].wait()
    st[1].wait()
    g = _gather(0)
    st = _stage(1)
    for i in range(N_CHUNKS):
        b = i % 2
        gn = None
        if i + 1 < N_CHUNKS:
            st[0].wait()
            st[1].wait()
            gn = _gather(i + 1)
        g.wait()
        pltpu.sync_copy(rows[b], acc_shared.at[di[b]], add=True)
        if i + 2 < N_CHUNKS:
            st = _stage(i + 2)
        g = gn

    plsc.subcore_barrier()

    @pl.when(sid < NS - 1)
    def _():
        pltpu.sync_copy(acc_shared.at[pl.ds(r0, ROW_CHUNK)],
                        acc_out.at[cid, pl.ds(r0, ROW_CHUNK)])

    @pl.when(sid == NS - 1)
    def _():
        pltpu.sync_copy(acc_shared.at[pl.ds(r0, 400)],
                        acc_out.at[cid, pl.ds(r0, 400)])


def _finish_body(acc_ref, h_ref, deg_ref, b_ref, out_ref):
    deg = deg_ref[0] + deg_ref[1] + 1.0
    dinv = lax.rsqrt(deg)
    acc = acc_ref[0] + acc_ref[1]
    out_ref[...] = jnp.maximum(
        dinv * acc + (dinv * dinv) * h_ref[...] + b_ref[...], 0.0)


def _finish(acc_parts, h, deg_parts, b1):
    R = N_NODES // 10
    return pl.pallas_call(
        _finish_body,
        grid=(10,),
        in_specs=[
            pl.BlockSpec((NC, R, HID), lambda i: (0, i, 0)),
            pl.BlockSpec((R, HID), lambda i: (i, 0)),
            pl.BlockSpec((NC, R, 1), lambda i: (0, i, 0)),
            pl.BlockSpec((1, HID), lambda i: (0, 0)),
        ],
        out_specs=pl.BlockSpec((R, HID), lambda i: (i, 0)),
        out_shape=jax.ShapeDtypeStruct((N_NODES, HID), jnp.float32),
    )(acc_parts, h, deg_parts.reshape(NC, N_NODES, 1), b1.reshape(1, HID))


def kernel(x, adj, W1, b1):
    adj = adj.astype(jnp.int32)
    h = _matmul(x, W1)
    deg_parts = _deg_kernel(adj)
    acc_parts = _agg_kernel(h, adj, deg_parts)
    return _finish(acc_parts, h, deg_parts, b1)


# speedup vs baseline: 1.1027x; 1.1027x over previous
import functools

import jax
import jax.numpy as jnp
from jax import lax
from jax.experimental import pallas as pl
from jax.experimental.pallas import tpu as pltpu
from jax.experimental.pallas import tpu_sc as plsc

N_NODES = 10000
N_EDGES = 640000
IN_CH = 116
HID = 64

NC = 2
NS = 16
NW = NC * NS
EDGES_PER_W = N_EDGES // NW
DEG_CHUNK = 2000
AGG_CHUNK = 400
ROW_CHUNK = 640

_mesh = plsc.VectorSubcoreMesh(core_axis_name="c", subcore_axis_name="s")
_sc_params = pltpu.CompilerParams(use_tc_tiling_on_sc=False)


def _mm_body(x_ref, w_ref, h_ref):
    h_ref[...] = jnp.dot(x_ref[...], w_ref[...],
                         preferred_element_type=jnp.float32)


def _matmul(x, W1):
    return pl.pallas_call(
        _mm_body,
        grid=(10,),
        in_specs=[
            pl.BlockSpec((N_NODES // 10, IN_CH), lambda i: (i, 0)),
            pl.BlockSpec((IN_CH, HID), lambda i: (0, 0)),
        ],
        out_specs=pl.BlockSpec((N_NODES // 10, HID), lambda i: (i, 0)),
        out_shape=jax.ShapeDtypeStruct((N_NODES, HID), jnp.float32),
    )(x, W1)


@functools.partial(
    pl.kernel,
    out_type=jax.ShapeDtypeStruct((NC, N_NODES), jnp.float32),
    mesh=_mesh,
    scratch_types=[
        pltpu.VMEM((DEG_CHUNK,), jnp.int32),
        pltpu.VMEM((DEG_CHUNK,), jnp.float32),
        pltpu.VMEM((DEG_CHUNK,), jnp.float32),
        pltpu.VMEM_SHARED((N_NODES,), jnp.float32),
    ],
    compiler_params=_sc_params,
)
def _deg_kernel(adj_hbm, deg_out, idx_v, ones_v, zeros_v, deg_shared):
    cid = lax.axis_index("c")
    sid = lax.axis_index("s")
    wid = cid * NS + sid

    @pl.loop(0, DEG_CHUNK, step=16)
    def _(i):
        ones_v[pl.ds(i, 16)] = jnp.full((16,), 1.0, jnp.float32)
        zeros_v[pl.ds(i, 16)] = jnp.zeros((16,), jnp.float32)

    @pl.when(sid == 0)
    def _():
        for j in range(N_NODES // DEG_CHUNK):
            pltpu.sync_copy(zeros_v, deg_shared.at[pl.ds(j * DEG_CHUNK,
                                                         DEG_CHUNK)])

    plsc.subcore_barrier()

    base = wid * EDGES_PER_W
    for i in range(EDGES_PER_W // DEG_CHUNK):
        pltpu.sync_copy(adj_hbm.at[1, pl.ds(base + i * DEG_CHUNK, DEG_CHUNK)],
                        idx_v)
        pltpu.sync_copy(ones_v, deg_shared.at[idx_v], add=True)

    plsc.subcore_barrier()

    @pl.when(sid == 0)
    def _():
        pltpu.sync_copy(deg_shared, deg_out.at[cid])


def _scale_body(h_ref, d0_ref, d1_ref, hs_ref, dinv_ref):
    deg = d0_ref[...] + d1_ref[...] + 1.0
    dinv = lax.rsqrt(deg)
    dinv_ref[...] = dinv
    hs_ref[...] = h_ref[...] * dinv


def _scale(h, deg0, deg1):
    R = N_NODES // 10
    return pl.pallas_call(
        _scale_body,
        grid=(10,),
        in_specs=[
            pl.BlockSpec((R, HID), lambda i: (i, 0)),
            pl.BlockSpec((R, 1), lambda i: (i, 0)),
            pl.BlockSpec((R, 1), lambda i: (i, 0)),
        ],
        out_specs=[
            pl.BlockSpec((R, HID), lambda i: (i, 0)),
            pl.BlockSpec((R, 1), lambda i: (i, 0)),
        ],
        out_shape=[
            jax.ShapeDtypeStruct((N_NODES, HID), jnp.float32),
            jax.ShapeDtypeStruct((N_NODES, 1), jnp.float32),
        ],
    )(h, deg0, deg1)


N_CHUNKS = EDGES_PER_W // AGG_CHUNK


@functools.partial(
    pl.kernel,
    out_type=jax.ShapeDtypeStruct((NC, N_NODES, HID), jnp.float32),
    mesh=_mesh,
    scratch_types=[
        pltpu.VMEM((AGG_CHUNK,), jnp.int32),
        pltpu.VMEM((AGG_CHUNK,), jnp.int32),
        pltpu.VMEM((AGG_CHUNK,), jnp.int32),
        pltpu.VMEM((AGG_CHUNK,), jnp.int32),
        pltpu.VMEM((AGG_CHUNK, HID), jnp.float32),
        pltpu.VMEM((80, HID), jnp.float32),
        pltpu.VMEM_SHARED((N_NODES, HID), jnp.float32),
        pltpu.VMEM_SHARED((N_NODES, HID), jnp.float32),
        pltpu.SemaphoreType.DMA,
        pltpu.SemaphoreType.DMA,
        pltpu.SemaphoreType.DMA,
    ],
    compiler_params=_sc_params,
)
def _agg_kernel(hs_hbm, adj_hbm, acc_out,
                si0_v, si1_v, di0_v, di1_v, rows_v, zb_v,
                acc_shared, hs_shared, sem_i0, sem_i1, sem_h):
    cid = lax.axis_index("c")
    sid = lax.axis_index("s")
    wid = cid * NS + sid

    r0 = sid * ROW_CHUNK

    @pl.loop(0, 80)
    def _(r):
        @pl.loop(0, HID, step=16)
        def _(c):
            zb_v[r, pl.ds(c, 16)] = jnp.zeros((16,), jnp.float32)

    @pl.when(sid < NS - 1)
    def _():
        hcp = pltpu.async_copy(hs_hbm.at[pl.ds(r0, ROW_CHUNK)],
                               hs_shared.at[pl.ds(r0, ROW_CHUNK)], sem_h)
        for j in range(ROW_CHUNK // 80):
            pltpu.sync_copy(zb_v, acc_shared.at[pl.ds(r0 + j * 80, 80)])
        hcp.wait()

    @pl.when(sid == NS - 1)
    def _():
        hcp = pltpu.async_copy(hs_hbm.at[pl.ds(r0, 400)],
                               hs_shared.at[pl.ds(r0, 400)], sem_h)
        for j in range(5):
            pltpu.sync_copy(zb_v, acc_shared.at[pl.ds(r0 + j * 80, 80)])
        hcp.wait()

    plsc.subcore_barrier()

    si = (si0_v, si1_v)
    di = (di0_v, di1_v)
    isems = (sem_i0, sem_i1)
    ebase = wid * EDGES_PER_W

    def _stage(j):
        b = j % 2
        off = ebase + j * AGG_CHUNK
        return (
            pltpu.async_copy(adj_hbm.at[0, pl.ds(off, AGG_CHUNK)], si[b],
                             isems[b]),
            pltpu.async_copy(adj_hbm.at[1, pl.ds(off, AGG_CHUNK)], di[b],
                             isems[b]),
        )

    st = _stage(0)
    for i in range(N_CHUNKS):
        b = i % 2
        st[0].wait()
        st[1].wait()
        if i + 1 < N_CHUNKS:
            st = _stage(i + 1)
        pltpu.sync_copy(hs_shared.at[si[b]], rows_v)
        pltpu.sync_copy(rows_v, acc_shared.at[di[b]], add=True)

    plsc.subcore_barrier()

    @pl.when(sid < NS - 1)
    def _():
        pltpu.sync_copy(acc_shared.at[pl.ds(r0, ROW_CHUNK)],
                        acc_out.at[cid, pl.ds(r0, ROW_CHUNK)])

    @pl.when(sid == NS - 1)
    def _():
        pltpu.sync_copy(acc_shared.at[pl.ds(r0, 400)],
                        acc_out.at[cid, pl.ds(r0, 400)])


def _finish_body(acc_ref, hs_ref, dinv_ref, b_ref, out_ref):
    a = acc_ref[0] + acc_ref[1] + hs_ref[...]
    out_ref[...] = jnp.maximum(a * dinv_ref[...] + b_ref[...], 0.0)


def _finish(acc_parts, hs, dinv, b1):
    R = N_NODES // 10
    return pl.pallas_call(
        _finish_body,
        grid=(10,),
        in_specs=[
            pl.BlockSpec((NC, R, HID), lambda i: (0, i, 0)),
            pl.BlockSpec((R, HID), lambda i: (i, 0)),
            pl.BlockSpec((R, 1), lambda i: (i, 0)),
            pl.BlockSpec((1, HID), lambda i: (0, 0)),
        ],
        out_specs=pl.BlockSpec((R, HID), lambda i: (i, 0)),
        out_shape=jax.ShapeDtypeStruct((N_NODES, HID), jnp.float32),
    )(acc_parts, hs, dinv, b1.reshape(1, HID))


def kernel(x, adj, W1, b1):
    adj = adj.astype(jnp.int32)
    h = _matmul(x, W1)
    deg_parts = _deg_kernel(adj)
    deg0 = deg_parts[0].reshape(N_NODES, 1)
    deg1 = deg_parts[1].reshape(N_NODES, 1)
    hs, dinv = _scale(h, deg0, deg1)
    acc_parts = _agg_kernel(hs, adj)
    return _finish(acc_parts, hs, dinv, b1)


# speedup vs baseline: 1.4225x; 1.2900x over previous
import functools

import jax
import jax.numpy as jnp
from jax import lax
from jax.experimental import pallas as pl
from jax.experimental.pallas import tpu as pltpu
from jax.experimental.pallas import tpu_sc as plsc

N_NODES = 10000
N_EDGES = 640000
IN_CH = 116
HID = 64

NC = 2
NS = 16
NW = NC * NS
EDGES_PER_W = N_EDGES // NW
DEG_CHUNK = 2000
AGG_CHUNK = 400
ROW_CHUNK = 640

_mesh = plsc.VectorSubcoreMesh(core_axis_name="c", subcore_axis_name="s")
_sc_params = pltpu.CompilerParams(use_tc_tiling_on_sc=False)


def _mm_body(x_ref, w_ref, h_ref):
    h_ref[...] = jnp.dot(x_ref[...], w_ref[...],
                         preferred_element_type=jnp.float32)


def _matmul(x, W1):
    return pl.pallas_call(
        _mm_body,
        grid=(10,),
        in_specs=[
            pl.BlockSpec((N_NODES // 10, IN_CH), lambda i: (i, 0)),
            pl.BlockSpec((IN_CH, HID), lambda i: (0, 0)),
        ],
        out_specs=pl.BlockSpec((N_NODES // 10, HID), lambda i: (i, 0)),
        out_shape=jax.ShapeDtypeStruct((N_NODES, HID), jnp.float32),
    )(x, W1)


@functools.partial(
    pl.kernel,
    out_type=jax.ShapeDtypeStruct((NC, N_NODES), jnp.float32),
    mesh=_mesh,
    scratch_types=[
        pltpu.VMEM((DEG_CHUNK,), jnp.int32),
        pltpu.VMEM((DEG_CHUNK,), jnp.float32),
        pltpu.VMEM((DEG_CHUNK,), jnp.float32),
        pltpu.VMEM_SHARED((N_NODES,), jnp.float32),
    ],
    compiler_params=_sc_params,
)
def _deg_kernel(adj_hbm, deg_out, idx_v, ones_v, zeros_v, deg_shared):
    cid = lax.axis_index("c")
    sid = lax.axis_index("s")
    wid = cid * NS + sid

    @pl.loop(0, DEG_CHUNK, step=16)
    def _(i):
        ones_v[pl.ds(i, 16)] = jnp.full((16,), 1.0, jnp.float32)
        zeros_v[pl.ds(i, 16)] = jnp.zeros((16,), jnp.float32)

    @pl.when(sid == 0)
    def _():
        for j in range(N_NODES // DEG_CHUNK):
            pltpu.sync_copy(zeros_v, deg_shared.at[pl.ds(j * DEG_CHUNK,
                                                         DEG_CHUNK)])

    plsc.subcore_barrier()

    base = wid * EDGES_PER_W
    for i in range(EDGES_PER_W // DEG_CHUNK):
        pltpu.sync_copy(adj_hbm.at[1, pl.ds(base + i * DEG_CHUNK, DEG_CHUNK)],
                        idx_v)
        pltpu.sync_copy(ones_v, deg_shared.at[idx_v], add=True)

    plsc.subcore_barrier()

    @pl.when(sid == 0)
    def _():
        pltpu.sync_copy(deg_shared, deg_out.at[cid])


def _scale_body(h_ref, d0_ref, d1_ref, hs_ref, dinv_ref):
    deg = d0_ref[...] + d1_ref[...] + 1.0
    dinv = lax.rsqrt(deg)
    dinv_ref[...] = dinv
    hs_ref[...] = h_ref[...] * dinv


def _scale(h, deg0, deg1):
    R = N_NODES // 10
    return pl.pallas_call(
        _scale_body,
        grid=(10,),
        in_specs=[
            pl.BlockSpec((R, HID), lambda i: (i, 0)),
            pl.BlockSpec((R, 1), lambda i: (i, 0)),
            pl.BlockSpec((R, 1), lambda i: (i, 0)),
        ],
        out_specs=[
            pl.BlockSpec((R, HID), lambda i: (i, 0)),
            pl.BlockSpec((R, 1), lambda i: (i, 0)),
        ],
        out_shape=[
            jax.ShapeDtypeStruct((N_NODES, HID), jnp.float32),
            jax.ShapeDtypeStruct((N_NODES, 1), jnp.float32),
        ],
    )(h, deg0, deg1)


N_CHUNKS = EDGES_PER_W // AGG_CHUNK


@functools.partial(
    pl.kernel,
    out_type=jax.ShapeDtypeStruct((NC, N_NODES, HID), jnp.float32),
    mesh=_mesh,
    scratch_types=[
        pltpu.VMEM((AGG_CHUNK,), jnp.int32),
        pltpu.VMEM((AGG_CHUNK,), jnp.int32),
        pltpu.VMEM((AGG_CHUNK,), jnp.int32),
        pltpu.VMEM((AGG_CHUNK,), jnp.int32),
        pltpu.VMEM((AGG_CHUNK, HID), jnp.float32),
        pltpu.VMEM((AGG_CHUNK, HID), jnp.float32),
        pltpu.VMEM((80, HID), jnp.float32),
        pltpu.VMEM_SHARED((N_NODES, HID), jnp.float32),
        pltpu.SemaphoreType.DMA,
        pltpu.SemaphoreType.DMA,
        pltpu.SemaphoreType.DMA,
        pltpu.SemaphoreType.DMA,
    ],
    compiler_params=_sc_params,
)
def _agg_kernel(hs_hbm, adj_hbm, acc_out,
                si0_v, si1_v, di0_v, di1_v, rows0_v, rows1_v, zb_v,
                acc_shared, sem_g0, sem_g1, sem_i0, sem_i1):
    cid = lax.axis_index("c")
    sid = lax.axis_index("s")
    wid = cid * NS + sid

    r0 = sid * ROW_CHUNK

    @pl.loop(0, 80)
    def _(r):
        @pl.loop(0, HID, step=16)
        def _(c):
            zb_v[r, pl.ds(c, 16)] = jnp.zeros((16,), jnp.float32)

    @pl.when(sid < NS - 1)
    def _():
        for j in range(ROW_CHUNK // 80):
            pltpu.sync_copy(zb_v, acc_shared.at[pl.ds(r0 + j * 80, 80)])

    @pl.when(sid == NS - 1)
    def _():
        for j in range(5):
            pltpu.sync_copy(zb_v, acc_shared.at[pl.ds(r0 + j * 80, 80)])

    plsc.subcore_barrier()

    si = (si0_v, si1_v)
    di = (di0_v, di1_v)
    rows = (rows0_v, rows1_v)
    gsems = (sem_g0, sem_g1)
    isems = (sem_i0, sem_i1)
    ebase = wid * EDGES_PER_W

    def _stage(j):
        b = j % 2
        off = ebase + j * AGG_CHUNK
        return (
            pltpu.async_copy(adj_hbm.at[0, pl.ds(off, AGG_CHUNK)], si[b],
                             isems[b]),
            pltpu.async_copy(adj_hbm.at[1, pl.ds(off, AGG_CHUNK)], di[b],
                             isems[b]),
        )

    def _gather(j):
        return pltpu.async_copy(hs_hbm.at[si[j % 2]], rows[j % 2],
                                gsems[j % 2])

    st = _stage(0)
    st[0].wait()
    st[1].wait()
    g = _gather(0)
    st = _stage(1)
    for i in range(N_CHUNKS):
        b = i % 2
        gn = None
        if i + 1 < N_CHUNKS:
            st[0].wait()
            st[1].wait()
            gn = _gather(i + 1)
        g.wait()
        pltpu.sync_copy(rows[b], acc_shared.at[di[b]], add=True)
        if i + 2 < N_CHUNKS:
            st = _stage(i + 2)
        g = gn

    plsc.subcore_barrier()

    @pl.when(sid < NS - 1)
    def _():
        pltpu.sync_copy(acc_shared.at[pl.ds(r0, ROW_CHUNK)],
                        acc_out.at[cid, pl.ds(r0, ROW_CHUNK)])

    @pl.when(sid == NS - 1)
    def _():
        pltpu.sync_copy(acc_shared.at[pl.ds(r0, 400)],
                        acc_out.at[cid, pl.ds(r0, 400)])


def _finish_body(acc_ref, hs_ref, dinv_ref, b_ref, out_ref):
    a = acc_ref[0] + acc_ref[1] + hs_ref[...]
    out_ref[...] = jnp.maximum(a * dinv_ref[...] + b_ref[...], 0.0)


def _finish(acc_parts, hs, dinv, b1):
    R = N_NODES // 10
    return pl.pallas_call(
        _finish_body,
        grid=(10,),
        in_specs=[
            pl.BlockSpec((NC, R, HID), lambda i: (0, i, 0)),
            pl.BlockSpec((R, HID), lambda i: (i, 0)),
            pl.BlockSpec((R, 1), lambda i: (i, 0)),
            pl.BlockSpec((1, HID), lambda i: (0, 0)),
        ],
        out_specs=pl.BlockSpec((R, HID), lambda i: (i, 0)),
        out_shape=jax.ShapeDtypeStruct((N_NODES, HID), jnp.float32),
    )(acc_parts, hs, dinv, b1.reshape(1, HID))


def kernel(x, adj, W1, b1):
    adj = adj.astype(jnp.int32)
    h = _matmul(x, W1)
    deg_parts = _deg_kernel(adj)
    deg0 = deg_parts[0].reshape(N_NODES, 1)
    deg1 = deg_parts[1].reshape(N_NODES, 1)
    hs, dinv = _scale(h, deg0, deg1)
    acc_parts = _agg_kernel(hs, adj)
    return _finish(acc_parts, hs, dinv, b1)
